# Initial kernel scaffold; baseline (speedup 1.0000x reference)
import jax
import jax.numpy as jnp
from jax.experimental import pallas as pl


def kernel(x, edge_index, edge_weights, Wz, bz, Wr, br, Wh, bh, lz_W, lz_b, lr_W, lr_b, lh_W, lh_b, att, lin_W, lin_b):
    raise NotImplementedError("write your pallas kernel here")


# speedup vs baseline: 18.6518x; 18.6518x over previous
import functools

import jax
import jax.numpy as jnp
from jax import lax
from jax.experimental import pallas as pl
from jax.experimental.pallas import tpu as pltpu
from jax.experimental.pallas import tpu_sc as plsc

N_NODES = 10000
N_PAD = 10240
CH = 32
HID = 16
T = 12
C_IN = 128
N_EDGES = 320000
K = 80
BN = 1024
SUB = N_PAD // 16

_mesh = functools.partial(
    plsc.VectorSubcoreMesh, core_axis_name="c", subcore_axis_name="s")


def _deg_body(col_hbm, w_hbm, zrow_hbm, deg_hbm, acc_sh, col_v, w_v):
    c = lax.axis_index("c")
    s = lax.axis_index("s")
    wid = c * 16 + s
    per_w = N_EDGES // 32
    pltpu.sync_copy(zrow_hbm, acc_sh.at[pl.ds(s * SUB, SUB)])
    plsc.subcore_barrier()

    def chunk(j, carry):
        base = wid * per_w + j * K
        pltpu.sync_copy(col_hbm.at[pl.ds(base, K)], col_v)
        pltpu.sync_copy(w_hbm.at[pl.ds(base, K)], w_v)
        pltpu.sync_copy(w_v, acc_sh.at[col_v], add=True)
        return carry

    lax.fori_loop(0, per_w // K, chunk, 0)
    plsc.subcore_barrier()
    pltpu.sync_copy(acc_sh.at[pl.ds(s * SUB, SUB)],
                    deg_hbm.at[c, pl.ds(s * SUB, SUB)])


def _degree(col, w):
    return pl.kernel(
        _deg_body,
        out_type=jax.ShapeDtypeStruct((2, N_PAD), jnp.float32),
        mesh=_mesh(),
        scratch_types=[
            pltpu.VMEM_SHARED((N_PAD,), jnp.float32),
            pltpu.VMEM((K,), jnp.int32),
            pltpu.VMEM((K,), jnp.float32),
        ],
    )(col, w, jnp.zeros((SUB,), jnp.float32))


def _proj_body(x_ref, w_ref, degp_ref, gp_ref, dis_ref):
    deg = degp_ref[0, :] + degp_ref[1, :] + 1.0
    dis = lax.rsqrt(deg)[:, None]
    g = jnp.dot(x_ref[...], w_ref[...], preferred_element_type=jnp.float32)
    gp_ref[...] = g * dis
    dis_ref[...] = dis


def _project(xflat, wbig, degp):
    grid = (N_PAD // BN,)
    return pl.pallas_call(
        _proj_body,
        grid=grid,
        in_specs=[
            pl.BlockSpec((BN, C_IN * T), lambda i: (i, 0)),
            pl.BlockSpec((C_IN * T, T * CH), lambda i: (0, 0)),
            pl.BlockSpec((2, BN), lambda i: (0, i)),
        ],
        out_specs=[
            pl.BlockSpec((BN, T * CH), lambda i: (i, 0)),
            pl.BlockSpec((BN, 1), lambda i: (i, 0)),
        ],
        out_shape=[
            jax.ShapeDtypeStruct((N_NODES, T * CH), jnp.float32),
            jax.ShapeDtypeStruct((N_NODES, 1), jnp.float32),
        ],
    )(xflat, wbig, degp)


def _scatter_body(gp_hbm, row_hbm, col_hbm, w_hbm, zblk_hbm, e_hbm,
                  acc_sh, row_v, col_v, w_v, idx_v, rows_v, sem):
    c = lax.axis_index("c")
    s = lax.axis_index("s")
    per_s = N_EDGES // 16

    def period(tt, carry):
        t = c * (T // 2) + tt
        pltpu.sync_copy(zblk_hbm, acc_sh.at[pl.ds(s * SUB, SUB), :])
        plsc.subcore_barrier()
        tvec = jnp.full((16,), t, jnp.int32)

        def chunk(j, inner):
            base = s * per_s + j * K
            pltpu.sync_copy(row_hbm.at[pl.ds(base, K)], row_v)
            pltpu.sync_copy(col_hbm.at[pl.ds(base, K)], col_v)
            pltpu.sync_copy(w_hbm.at[pl.ds(base, K)], w_v)
            for i in range(K // 16):
                sl = pl.ds(i * 16, 16)
                idx_v[sl] = row_v[sl] * T + tvec
            pltpu.async_copy(gp_hbm.at[idx_v], rows_v, sem).wait()
            for i in range(K // 16):
                wv16 = w_v[pl.ds(i * 16, 16)]
                for m in range(16):
                    e = i * 16 + m
                    wsp = jnp.full((16,), wv16[m])
                    rows_v[e, pl.ds(0, 16)] = rows_v[e, pl.ds(0, 16)] * wsp
                    rows_v[e, pl.ds(16, 16)] = rows_v[e, pl.ds(16, 16)] * wsp
            pltpu.sync_copy(rows_v, acc_sh.at[col_v], add=True)
            return inner

        lax.fori_loop(0, per_s // K, chunk, 0)
        plsc.subcore_barrier()
        pltpu.sync_copy(acc_sh.at[pl.ds(s * SUB, SUB), :],
                        e_hbm.at[t, pl.ds(s * SUB, SUB), :])
        plsc.subcore_barrier()
        return carry

    lax.fori_loop(0, T // 2, period, 0)


def _edge_scatter(gp12, row, col, w):
    return pl.kernel(
        _scatter_body,
        out_type=jax.ShapeDtypeStruct((T, N_PAD, CH), jnp.float32),
        mesh=_mesh(),
        compiler_params=pltpu.CompilerParams(use_tc_tiling_on_sc=False),
        scratch_types=[
            pltpu.VMEM_SHARED((N_PAD, CH), jnp.float32),
            pltpu.VMEM((K,), jnp.int32),
            pltpu.VMEM((K,), jnp.int32),
            pltpu.VMEM((K,), jnp.float32),
            pltpu.VMEM((K,), jnp.int32),
            pltpu.VMEM((K, CH), jnp.float32),
            pltpu.SemaphoreType.DMA,
        ],
    )(gp12, row, col, w, jnp.zeros((SUB, CH), jnp.float32))


def _fin_body(e_ref, gp_ref, dis_ref, att_ref, bz_ref, bh_ref, lw_ref,
              lb_ref, o_ref):
    att = att_ref[0, :]
    p = jnp.exp(att - jnp.max(att))
    probs = p / jnp.sum(p)
    dis = dis_ref[...]
    acc = jnp.zeros((BN, HID), jnp.float32)
    for t in range(T):
        s_t = dis * (e_ref[t, :, :] + gp_ref[:, t * CH:(t + 1) * CH])
        z = jax.nn.sigmoid(s_t[:, :HID] + bz_ref[0, :])
        h = jnp.tanh(s_t[:, HID:] + bh_ref[0, :])
        acc = acc + probs[t] * ((1.0 - z) * h)
    o_ref[...] = (jnp.dot(jnp.maximum(acc, 0.0), lw_ref[...],
                          preferred_element_type=jnp.float32) + lb_ref[0, :])


def _finish(e, gp, dis, att, bz2, bh2, lin_W, lin_b):
    grid = (N_PAD // BN,)
    return pl.pallas_call(
        _fin_body,
        grid=grid,
        in_specs=[
            pl.BlockSpec((T, BN, CH), lambda i: (0, i, 0)),
            pl.BlockSpec((BN, T * CH), lambda i: (i, 0)),
            pl.BlockSpec((BN, 1), lambda i: (i, 0)),
            pl.BlockSpec((1, T), lambda i: (0, 0)),
            pl.BlockSpec((1, HID), lambda i: (0, 0)),
            pl.BlockSpec((1, HID), lambda i: (0, 0)),
            pl.BlockSpec((HID, T), lambda i: (0, 0)),
            pl.BlockSpec((1, T), lambda i: (0, 0)),
        ],
        out_specs=pl.BlockSpec((BN, T), lambda i: (i, 0)),
        out_shape=jax.ShapeDtypeStruct((N_NODES, T), jnp.float32),
    )(e, gp, dis, att, bz2, bh2, lin_W, lin_b)


def kernel(x, edge_index, edge_weights, Wz, bz, Wr, br, Wh, bh,
           lz_W, lz_b, lr_W, lr_b, lh_W, lh_b, att, lin_W, lin_b):
    wzf = Wz @ lz_W[:HID]
    whf = Wh @ lh_W[:HID]
    bz2 = (bz @ lz_W[:HID] + lz_b).reshape(1, HID)
    bh2 = (bh @ lh_W[:HID] + lh_b).reshape(1, HID)
    wcomb = jnp.concatenate([wzf, whf], axis=1)
    wbig = jnp.einsum("tu,ck->ctuk", jnp.eye(T, dtype=x.dtype),
                      wcomb).reshape(C_IN * T, T * CH)

    row = edge_index[0]
    col = edge_index[1]
    degp = _degree(col, edge_weights)
    gp, dis = _project(x.reshape(N_NODES, C_IN * T), wbig, degp)
    e = _edge_scatter(gp.reshape(N_NODES * T, CH), row, col, edge_weights)
    return _finish(e, gp, dis, att.reshape(1, T), bz2, bh2, lin_W,
                   lin_b.reshape(1, T))


# speedup vs baseline: 59.3174x; 3.1803x over previous
import functools

import jax
import jax.numpy as jnp
from jax import lax
from jax.experimental import pallas as pl
from jax.experimental.pallas import tpu as pltpu
from jax.experimental.pallas import tpu_sc as plsc

N_NODES = 10000
N_PAD = 10240
CH = 32
GRP = 4
NG = 3
GW = GRP * CH
HID = 16
T = 12
C_IN = 128
N_EDGES = 320000
K = 80
BN = 1024
SUB = N_PAD // 16

_mesh = functools.partial(
    plsc.VectorSubcoreMesh, core_axis_name="c", subcore_axis_name="s")


def _deg_body(col_hbm, w_hbm, zrow_hbm, deg_hbm, acc_sh, col_v, w_v):
    c = lax.axis_index("c")
    s = lax.axis_index("s")
    wid = c * 16 + s
    per_w = N_EDGES // 32
    pltpu.sync_copy(zrow_hbm, acc_sh.at[pl.ds(s * SUB, SUB)])
    plsc.subcore_barrier()

    def chunk(j, carry):
        base = wid * per_w + j * K
        pltpu.sync_copy(col_hbm.at[pl.ds(base, K)], col_v)
        pltpu.sync_copy(w_hbm.at[pl.ds(base, K)], w_v)
        pltpu.sync_copy(w_v, acc_sh.at[col_v], add=True)
        return carry

    lax.fori_loop(0, per_w // K, chunk, 0)
    plsc.subcore_barrier()
    pltpu.sync_copy(acc_sh.at[pl.ds(s * SUB, SUB)],
                    deg_hbm.at[c, pl.ds(s * SUB, SUB)])


def _degree(col, w):
    return pl.kernel(
        _deg_body,
        out_type=jax.ShapeDtypeStruct((2, N_PAD), jnp.float32),
        mesh=_mesh(),
        scratch_types=[
            pltpu.VMEM_SHARED((N_PAD,), jnp.float32),
            pltpu.VMEM((K,), jnp.int32),
            pltpu.VMEM((K,), jnp.float32),
        ],
    )(col, w, jnp.zeros((SUB,), jnp.float32))


def _proj_body(x_ref, w_ref, degp_ref, gp_ref, dis_ref):
    deg = degp_ref[0, :] + degp_ref[1, :] + 1.0
    dis = lax.rsqrt(deg)[:, None]
    g = jnp.dot(x_ref[...], w_ref[...], preferred_element_type=jnp.float32)
    g = g * dis
    for q in range(NG):
        gp_ref[q] = g[:, q * GW:(q + 1) * GW]
    dis_ref[...] = dis


def _project(xflat, wbig, degp):
    grid = (N_PAD // BN,)
    return pl.pallas_call(
        _proj_body,
        grid=grid,
        in_specs=[
            pl.BlockSpec((BN, C_IN * T), lambda i: (i, 0)),
            pl.BlockSpec((C_IN * T, T * CH), lambda i: (0, 0)),
            pl.BlockSpec((2, BN), lambda i: (0, i)),
        ],
        out_specs=[
            pl.BlockSpec((NG, BN, GW), lambda i: (0, i, 0)),
            pl.BlockSpec((BN, 1), lambda i: (i, 0)),
        ],
        out_shape=[
            jax.ShapeDtypeStruct((NG, N_NODES, GW), jnp.float32),
            jax.ShapeDtypeStruct((N_NODES, 1), jnp.float32),
        ],
    )(xflat, wbig, degp)


def _scatter_body(gp_hbm, row_hbm, col_hbm, w_hbm, zblk_hbm, e_hbm,
                  acc_sh, row_v, col_v, w_v, idx_v, rows_v, sem0, sem1):
    c = lax.axis_index("c")
    s = lax.axis_index("s")
    per_s = N_EDGES // 32
    nchunk = per_s // K

    sems = (sem0, sem1)

    def load_and_fire(j, slot, qvec):
        base = (c * 16 + s) * per_s + j * K
        pltpu.sync_copy(row_hbm.at[pl.ds(base, K)], row_v.at[slot])
        pltpu.sync_copy(col_hbm.at[pl.ds(base, K)], col_v.at[slot])
        pltpu.sync_copy(w_hbm.at[pl.ds(base, K)], w_v.at[slot])
        for i in range(K // 16):
            sl = pl.ds(i * 16, 16)
            idx_v[slot, sl] = row_v[slot, sl] + qvec
        pltpu.async_copy(gp_hbm.at[idx_v.at[slot]], rows_v.at[slot], sems[slot])

    def drain(slot):
        pltpu.make_async_copy(gp_hbm.at[idx_v.at[slot]], rows_v.at[slot],
                              sems[slot]).wait()

    def scale_and_scatter(slot):
        for i in range(K // 16):
            wv16 = w_v[slot, pl.ds(i * 16, 16)]
            for m in range(16):
                e = i * 16 + m
                wsp = jnp.full((16,), wv16[m])
                for h in range(GW // 16):
                    sl2 = pl.ds(h * 16, 16)
                    rows_v[slot, e, sl2] = rows_v[slot, e, sl2] * wsp
        pltpu.sync_copy(rows_v.at[slot], acc_sh.at[col_v.at[slot]], add=True)

    def group(q, carry):
        pltpu.sync_copy(zblk_hbm, acc_sh.at[pl.ds(s * SUB, SUB), :])
        plsc.subcore_barrier()
        qvec = jnp.full((16,), q * N_NODES, jnp.int32)

        load_and_fire(0, 0, qvec)

        def chunk2(jj, inner):
            j = jj * 2
            load_and_fire(j + 1, 1, qvec)
            drain(0)
            scale_and_scatter(0)
            @pl.when(jj * 2 + 2 < nchunk)
            def _():
                load_and_fire(j + 2, 0, qvec)
            drain(1)
            scale_and_scatter(1)
            return inner

        lax.fori_loop(0, nchunk // 2, chunk2, 0)
        if nchunk % 2 == 1:
            drain(0)
            scale_and_scatter(0)
        plsc.subcore_barrier()
        pltpu.sync_copy(acc_sh.at[pl.ds(s * SUB, SUB), :],
                        e_hbm.at[c, q, pl.ds(s * SUB, SUB), :])
        plsc.subcore_barrier()
        return carry

    lax.fori_loop(0, NG, group, 0)


def _edge_scatter(gpg, row, col, w):
    return pl.kernel(
        _scatter_body,
        out_type=jax.ShapeDtypeStruct((2, NG, N_PAD, GW), jnp.float32),
        mesh=_mesh(),
        compiler_params=pltpu.CompilerParams(use_tc_tiling_on_sc=False),
        scratch_types=[
            pltpu.VMEM_SHARED((N_PAD, GW), jnp.float32),
            pltpu.VMEM((2, K), jnp.int32),
            pltpu.VMEM((2, K), jnp.int32),
            pltpu.VMEM((2, K), jnp.float32),
            pltpu.VMEM((2, K), jnp.int32),
            pltpu.VMEM((2, K, GW), jnp.float32),
            pltpu.SemaphoreType.DMA,
            pltpu.SemaphoreType.DMA,
        ],
    )(gpg, row, col, w, jnp.zeros((SUB, GW), jnp.float32))


def _fin_body(e_ref, gp_ref, dis_ref, att_ref, bz_ref, bh_ref, lw_ref,
              lb_ref, o_ref):
    att = att_ref[0, :]
    p = jnp.exp(att - jnp.max(att))
    probs = p / jnp.sum(p)
    dis = dis_ref[...]
    acc = jnp.zeros((BN, HID), jnp.float32)
    for t in range(T):
        q, r = divmod(t, GRP)
        sl = pl.ds(r * CH, CH)
        e_t = e_ref[0, q, :, sl] + e_ref[1, q, :, sl]
        s_t = dis * (e_t + gp_ref[q, :, sl])
        z = jax.nn.sigmoid(s_t[:, :HID] + bz_ref[0, :])
        h = jnp.tanh(s_t[:, HID:] + bh_ref[0, :])
        acc = acc + probs[t] * ((1.0 - z) * h)
    o_ref[...] = (jnp.dot(jnp.maximum(acc, 0.0), lw_ref[...],
                          preferred_element_type=jnp.float32) + lb_ref[0, :])


def _finish(e, gp, dis, att, bz2, bh2, lin_W, lin_b):
    grid = (N_PAD // BN,)
    return pl.pallas_call(
        _fin_body,
        grid=grid,
        in_specs=[
            pl.BlockSpec((2, NG, BN, GW), lambda i: (0, 0, i, 0)),
            pl.BlockSpec((NG, BN, GW), lambda i: (0, i, 0)),
            pl.BlockSpec((BN, 1), lambda i: (i, 0)),
            pl.BlockSpec((1, T), lambda i: (0, 0)),
            pl.BlockSpec((1, HID), lambda i: (0, 0)),
            pl.BlockSpec((1, HID), lambda i: (0, 0)),
            pl.BlockSpec((HID, T), lambda i: (0, 0)),
            pl.BlockSpec((1, T), lambda i: (0, 0)),
        ],
        out_specs=pl.BlockSpec((BN, T), lambda i: (i, 0)),
        out_shape=jax.ShapeDtypeStruct((N_NODES, T), jnp.float32),
    )(e, gp, dis, att, bz2, bh2, lin_W, lin_b)


def kernel(x, edge_index, edge_weights, Wz, bz, Wr, br, Wh, bh,
           lz_W, lz_b, lr_W, lr_b, lh_W, lh_b, att, lin_W, lin_b):
    wzf = Wz @ lz_W[:HID]
    whf = Wh @ lh_W[:HID]
    bz2 = (bz @ lz_W[:HID] + lz_b).reshape(1, HID)
    bh2 = (bh @ lh_W[:HID] + lh_b).reshape(1, HID)
    wcomb = jnp.concatenate([wzf, whf], axis=1)
    wbig = jnp.einsum("tu,ck->ctuk", jnp.eye(T, dtype=x.dtype),
                      wcomb).reshape(C_IN * T, T * CH)

    row = edge_index[0]
    col = edge_index[1]
    degp = _degree(col, edge_weights)
    gp, dis = _project(x.reshape(N_NODES, C_IN * T), wbig, degp)
    e = _edge_scatter(gp.reshape(NG * N_NODES, GW), row, col, edge_weights)
    return _finish(e, gp, dis, att.reshape(1, T), bz2, bh2, lin_W,
                   lin_b.reshape(1, T))
